# Initial kernel scaffold; baseline (speedup 1.0000x reference)
import jax
import jax.numpy as jnp
from jax.experimental import pallas as pl


def kernel(item_tensors, table):
    raise NotImplementedError("write your pallas kernel here")


# speedup vs baseline: 1.1602x; 1.1602x over previous
import functools

import jax
import jax.numpy as jnp
from jax import lax
from jax.experimental import pallas as pl
from jax.experimental.pallas import tpu as pltpu
from jax.experimental.pallas import tpu_sc as plsc

VOCAB = 1000000
EMBED = 32
B = 16384
L = 50
LPAD = 56
NC = 2
NS = 16
NW = NC * NS
RW = B // NW
ROWS_PER_CHUNK = 2
CHUNK = ROWS_PER_CHUNK * LPAD
NCH = RW // ROWS_PER_CHUNK

_mesh = plsc.VectorSubcoreMesh(
    core_axis_name="c", subcore_axis_name="s", num_cores=NC, num_subcores=NS
)


@functools.partial(
    pl.kernel,
    out_type=jax.ShapeDtypeStruct((B * EMBED,), jnp.float32),
    mesh=_mesh,
    scratch_types=[
        pltpu.VMEM((NCH, CHUNK), jnp.int32),
        pltpu.VMEM((CHUNK, EMBED), jnp.float32),
        pltpu.VMEM((CHUNK, EMBED), jnp.float32),
        pltpu.VMEM((RW * EMBED,), jnp.float32),
        pltpu.SemaphoreType.DMA,
        pltpu.SemaphoreType.DMA,
    ],
    compiler_params=pltpu.CompilerParams(use_tc_tiling_on_sc=False),
)
def _sc_embed_mean(table_hbm, idx_hbm, out_hbm, idx_v, g0, g1, out_v, s0, s1):
    wid = lax.axis_index("c") * NS + lax.axis_index("s")
    pltpu.sync_copy(idx_hbm.at[wid], idx_v)

    def start(c, g, sem):
        pltpu.async_copy(table_hbm.at[idx_v.at[c]], g, sem)

    def wait(g, sem):
        pltpu.make_async_copy(table_hbm.at[idx_v.at[0]], g, sem).wait()

    scale = jnp.float32(1.0 / L)

    def process(c, g):
        out_base = c * (ROWS_PER_CHUNK * EMBED)
        for r in range(ROWS_PER_CHUNK):
            b0 = r * LPAD
            acc0 = g[b0, pl.ds(0, 16)]
            acc1 = g[b0, pl.ds(16, 16)]
            for j in range(1, L):
                acc0 = acc0 + g[b0 + j, pl.ds(0, 16)]
                acc1 = acc1 + g[b0 + j, pl.ds(16, 16)]
            out_v[pl.ds(out_base + r * EMBED, 16)] = acc0 * scale
            out_v[pl.ds(out_base + r * EMBED + 16, 16)] = acc1 * scale

    start(0, g0, s0)
    start(1, g1, s1)

    @pl.loop(0, NCH - 2, step=2)
    def _(c):
        wait(g0, s0)
        process(c, g0)
        start(c + 2, g0, s0)
        wait(g1, s1)
        process(c + 1, g1)
        start(c + 3, g1, s1)

    wait(g0, s0)
    process(NCH - 2, g0)
    wait(g1, s1)
    process(NCH - 1, g1)

    pltpu.sync_copy(out_v, out_hbm.at[pl.ds(wid * (RW * EMBED), RW * EMBED)])


def kernel(item_tensors, table):
    idx = jnp.pad(item_tensors, ((0, 0), (0, LPAD - L)))
    idx = idx.reshape(NW, NCH, CHUNK)
    out = _sc_embed_mean(table, idx)
    return out.reshape(B, EMBED)
